# baseline (speedup 1.0000x reference)
import jax
import jax.numpy as jnp
from jax import lax
from jax.experimental import pallas as pl
from jax.experimental.pallas import tpu as pltpu
from jax.experimental.pallas import tpu_sc as plsc

N = 10000
D = 128
E = 320000
NC = 2
NS = 16
HALF = N // NC
EPT = E // NS
K = 128
NBLK = 157
EPT_PAD = NBLK * K
ACC_ROWS = 5128
STRIPE = 320
ZR = 80
TRASH = HALF
DEG_ROWS = 5376
DEG_PAD = 5120


def _remap_dst(dst_s, c, s):
  lo = c * HALF
  trash_base = TRASH + s * 8

  def remap(i, carry):
    for j in range(K // 16):
      v = dst_s[i, pl.ds(j * 16, 16)]
      local = v - lo
      owned = (local >= 0) & (local < HALF)
      trash = trash_base + (v & 7)
      dst_s[i, pl.ds(j * 16, 16)] = jnp.where(owned, local, trash)
    return carry
  lax.fori_loop(0, NBLK, remap, 0)


def _make_agg():
  mesh = plsc.VectorSubcoreMesh(core_axis_name="c", subcore_axis_name="s")

  def body(h_hbm, src_hbm, dst_hbm, out_sum, src_s, dst_s, rows0, zbuf,
           acc_sh, sem_a):
    c = lax.axis_index("c")
    s = lax.axis_index("s")
    lo = c * HALF

    def zero_zbuf(i, carry):
      for j in range(D // 16):
        zbuf[i, pl.ds(j * 16, 16)] = jnp.zeros((16,), jnp.float32)
      return carry
    lax.fori_loop(0, ZR, zero_zbuf, 0)
    for r in range(STRIPE // ZR):
      pltpu.sync_copy(zbuf, acc_sh.at[pl.ds(s * STRIPE + r * ZR, ZR)])

    @pl.when(s == 0)
    def _():
      pltpu.sync_copy(zbuf.at[pl.ds(0, ACC_ROWS - NS * STRIPE)],
                      acc_sh.at[pl.ds(NS * STRIPE, ACC_ROWS - NS * STRIPE)])

    pltpu.sync_copy(src_hbm.at[s], src_s)
    pltpu.sync_copy(dst_hbm.at[s], dst_s)
    _remap_dst(dst_s, c, s)
    plsc.subcore_barrier()

    def step(i, carry):
      pltpu.async_copy(h_hbm.at[src_s.at[i]], rows0, sem_a).wait()
      pltpu.sync_copy(rows0, acc_sh.at[dst_s.at[i]], add=True)
      return carry
    lax.fori_loop(0, NBLK, step, 0)

    plsc.subcore_barrier()
    @pl.when(s < NS - 1)
    def _():
      pltpu.sync_copy(acc_sh.at[pl.ds(s * STRIPE, STRIPE)],
                      out_sum.at[pl.ds(lo + s * STRIPE, STRIPE)])

    @pl.when(s == NS - 1)
    def _():
      last = HALF - (NS - 1) * STRIPE
      pltpu.sync_copy(acc_sh.at[pl.ds((NS - 1) * STRIPE, last)],
                      out_sum.at[pl.ds(lo + (NS - 1) * STRIPE, last)])

  return pl.kernel(
      body,
      out_type=jax.ShapeDtypeStruct((N, D), jnp.float32),
      mesh=mesh,
      scratch_types=[
          pltpu.VMEM((NBLK, K), jnp.int32),
          pltpu.VMEM((NBLK, K), jnp.int32),
          pltpu.VMEM((K, D), jnp.float32),
          pltpu.VMEM((ZR, D), jnp.float32),
          pltpu.VMEM_SHARED((ACC_ROWS, D), jnp.float32),
          pltpu.SemaphoreType.DMA,
      ])


def _make_deg():
  mesh = plsc.VectorSubcoreMesh(core_axis_name="c", subcore_axis_name="s")

  def body(dst_hbm, out_deg, dst_s, ones_s, zdeg_s, deg_sh):
    c = lax.axis_index("c")
    s = lax.axis_index("s")
    lo = c * HALF

    def zero_zdeg(i, carry):
      zdeg_s[pl.ds(i * 16, 16)] = jnp.zeros((16,), jnp.float32)
      return carry
    lax.fori_loop(0, DEG_ROWS // 16, zero_zdeg, 0)
    for j in range(K // 16):
      ones_s[pl.ds(j * 16, 16)] = jnp.ones((16,), jnp.float32)

    @pl.when(s == 0)
    def _():
      pltpu.sync_copy(zdeg_s, deg_sh)

    pltpu.sync_copy(dst_hbm.at[s], dst_s)
    _remap_dst(dst_s, c, s)
    plsc.subcore_barrier()

    def step(i, carry):
      pltpu.sync_copy(ones_s, deg_sh.at[dst_s.at[i]], add=True)
      return carry
    lax.fori_loop(0, NBLK, step, 0)

    plsc.subcore_barrier()

    @pl.when(s == 0)
    def _():
      pltpu.sync_copy(deg_sh.at[pl.ds(0, HALF)], zdeg_s.at[pl.ds(0, HALF)])
      pltpu.sync_copy(zdeg_s.at[pl.ds(0, HALF)], out_deg.at[pl.ds(lo, HALF)])

  return pl.kernel(
      body,
      out_type=jax.ShapeDtypeStruct((N,), jnp.float32),
      mesh=mesh,
      scratch_types=[
          pltpu.VMEM((NBLK, K), jnp.int32),
          pltpu.VMEM((K,), jnp.float32),
          pltpu.VMEM((DEG_ROWS,), jnp.float32),
          pltpu.VMEM_SHARED((DEG_ROWS,), jnp.float32),
      ])


_agg = _make_agg()
_deg = _make_deg()


BN = 1000


def _make_layer(do_ln: bool):
  def body(*refs):
    if do_ln:
      sum_ref, deg_ref, h_ref, wl_ref, bl_ref, wr_ref, g_ref, b_ref, o_ref = refs
    else:
      sum_ref, deg_ref, h_ref, wl_ref, bl_ref, wr_ref, o_ref = refs
    mean = sum_ref[...] / jnp.clip(deg_ref[...], 1.0, None)
    out = (jnp.dot(mean, wl_ref[...], preferred_element_type=jnp.float32)
           + bl_ref[...]
           + jnp.dot(h_ref[...], wr_ref[...], preferred_element_type=jnp.float32))
    out = jnp.maximum(out, 0.0)
    if do_ln:
      mu = jnp.mean(out, axis=1, keepdims=True)
      var = jnp.mean((out - mu) ** 2, axis=1, keepdims=True)
      out = (out - mu) / jnp.sqrt(var + 1e-5) * g_ref[...] + b_ref[...]
    o_ref[...] = out

  in_specs = [
      pl.BlockSpec((BN, D), lambda i: (i, 0)),
      pl.BlockSpec((BN, 1), lambda i: (i, 0)),
      pl.BlockSpec((BN, D), lambda i: (i, 0)),
      pl.BlockSpec((D, D), lambda i: (0, 0)),
      pl.BlockSpec((1, D), lambda i: (0, 0)),
      pl.BlockSpec((D, D), lambda i: (0, 0)),
  ]
  if do_ln:
    in_specs += [pl.BlockSpec((1, D), lambda i: (0, 0))] * 2
  return pl.pallas_call(
      body,
      grid=(N // BN,),
      in_specs=in_specs,
      out_specs=pl.BlockSpec((BN, D), lambda i: (i, 0)),
      out_shape=jax.ShapeDtypeStruct((N, D), jnp.float32),
  )


_layer_plain = _make_layer(False)
_layer_ln = _make_layer(True)


def kernel(x, edge_index, Wl1, bl1, Wr1, Wl2, bl2, Wr2, ln_g, ln_b):
  pad = EPT_PAD - EPT
  src = jnp.concatenate(
      [edge_index[0].astype(jnp.int32).reshape(NS, EPT),
       jnp.zeros((NS, pad), jnp.int32)], axis=1).reshape(NS, NBLK, K)
  dst = jnp.concatenate(
      [edge_index[1].astype(jnp.int32).reshape(NS, EPT),
       jnp.full((NS, pad), -1, jnp.int32)], axis=1).reshape(NS, NBLK, K)
  deg2 = _deg(dst).reshape(N, 1)
  sums1 = _agg(x, src, dst)
  h1 = _layer_plain(sums1, deg2, x, Wl1.T, bl1.reshape(1, D), Wr1.T)
  sums2 = _agg(h1, src, dst)
  return _layer_ln(sums2, deg2, h1, Wl2.T, bl2.reshape(1, D), Wr2.T,
                   ln_g.reshape(1, D), ln_b.reshape(1, D))


# speedup vs baseline: 1.0758x; 1.0758x over previous
import jax
import jax.numpy as jnp
from jax import lax
from jax.experimental import pallas as pl
from jax.experimental.pallas import tpu as pltpu
from jax.experimental.pallas import tpu_sc as plsc

N = 10000
D = 128
E = 320000
NC = 2
NS = 16
HALF = N // NC
EPT = E // NS
K = 80
NBLK = EPT // K
ACC_ROWS = 5128
STRIPE = 320
ZR = 80
TRASH = HALF
DEG_ROWS = 5376
DEG_PAD = 5120


def _remap_dst(dst_s, c, s):
  lo = c * HALF
  trash_base = TRASH + s * 8

  def remap(i, carry):
    for j in range(K // 16):
      v = dst_s[i, pl.ds(j * 16, 16)]
      local = v - lo
      owned = (local >= 0) & (local < HALF)
      trash = trash_base + (v & 7)
      dst_s[i, pl.ds(j * 16, 16)] = jnp.where(owned, local, trash)
    return carry
  lax.fori_loop(0, NBLK, remap, 0)


def _make_agg():
  mesh = plsc.VectorSubcoreMesh(core_axis_name="c", subcore_axis_name="s")

  def body(h_hbm, src_hbm, dst_hbm, out_sum, src_s, dst_s, rows0, zbuf,
           acc_sh, sem_a):
    c = lax.axis_index("c")
    s = lax.axis_index("s")
    lo = c * HALF

    def zero_zbuf(i, carry):
      for j in range(D // 16):
        zbuf[i, pl.ds(j * 16, 16)] = jnp.zeros((16,), jnp.float32)
      return carry
    lax.fori_loop(0, ZR, zero_zbuf, 0)
    for r in range(STRIPE // ZR):
      pltpu.sync_copy(zbuf, acc_sh.at[pl.ds(s * STRIPE + r * ZR, ZR)])

    @pl.when(s == 0)
    def _():
      pltpu.sync_copy(zbuf.at[pl.ds(0, ACC_ROWS - NS * STRIPE)],
                      acc_sh.at[pl.ds(NS * STRIPE, ACC_ROWS - NS * STRIPE)])

    pltpu.sync_copy(src_hbm.at[s], src_s)
    pltpu.sync_copy(dst_hbm.at[s], dst_s)
    _remap_dst(dst_s, c, s)
    plsc.subcore_barrier()

    def step(i, carry):
      pltpu.async_copy(h_hbm.at[src_s.at[i]], rows0, sem_a).wait()
      pltpu.sync_copy(rows0, acc_sh.at[dst_s.at[i]], add=True)
      return carry
    lax.fori_loop(0, NBLK, step, 0)

    plsc.subcore_barrier()
    @pl.when(s < NS - 1)
    def _():
      pltpu.sync_copy(acc_sh.at[pl.ds(s * STRIPE, STRIPE)],
                      out_sum.at[pl.ds(lo + s * STRIPE, STRIPE)])

    @pl.when(s == NS - 1)
    def _():
      last = HALF - (NS - 1) * STRIPE
      pltpu.sync_copy(acc_sh.at[pl.ds((NS - 1) * STRIPE, last)],
                      out_sum.at[pl.ds(lo + (NS - 1) * STRIPE, last)])

  return pl.kernel(
      body,
      out_type=jax.ShapeDtypeStruct((N, D), jnp.float32),
      mesh=mesh,
      scratch_types=[
          pltpu.VMEM((NBLK, K), jnp.int32),
          pltpu.VMEM((NBLK, K), jnp.int32),
          pltpu.VMEM((K, D), jnp.float32),
          pltpu.VMEM((ZR, D), jnp.float32),
          pltpu.VMEM_SHARED((ACC_ROWS, D), jnp.float32),
          pltpu.SemaphoreType.DMA,
      ])


def _make_deg():
  mesh = plsc.VectorSubcoreMesh(core_axis_name="c", subcore_axis_name="s")

  def body(dst_hbm, out_deg, dst_s, ones_s, zdeg_s, deg_sh):
    c = lax.axis_index("c")
    s = lax.axis_index("s")
    lo = c * HALF

    def zero_zdeg(i, carry):
      zdeg_s[pl.ds(i * 16, 16)] = jnp.zeros((16,), jnp.float32)
      return carry
    lax.fori_loop(0, DEG_ROWS // 16, zero_zdeg, 0)
    for j in range(K // 16):
      ones_s[pl.ds(j * 16, 16)] = jnp.ones((16,), jnp.float32)

    @pl.when(s == 0)
    def _():
      pltpu.sync_copy(zdeg_s, deg_sh)

    pltpu.sync_copy(dst_hbm.at[s], dst_s)
    _remap_dst(dst_s, c, s)
    plsc.subcore_barrier()

    def step(i, carry):
      pltpu.sync_copy(ones_s, deg_sh.at[dst_s.at[i]], add=True)
      return carry
    lax.fori_loop(0, NBLK, step, 0)

    plsc.subcore_barrier()

    @pl.when(s == 0)
    def _():
      pltpu.sync_copy(deg_sh.at[pl.ds(0, HALF)], zdeg_s.at[pl.ds(0, HALF)])
      pltpu.sync_copy(zdeg_s.at[pl.ds(0, HALF)], out_deg.at[pl.ds(lo, HALF)])

  return pl.kernel(
      body,
      out_type=jax.ShapeDtypeStruct((N,), jnp.float32),
      mesh=mesh,
      scratch_types=[
          pltpu.VMEM((NBLK, K), jnp.int32),
          pltpu.VMEM((K,), jnp.float32),
          pltpu.VMEM((DEG_ROWS,), jnp.float32),
          pltpu.VMEM_SHARED((DEG_ROWS,), jnp.float32),
      ])


_agg = _make_agg()
_deg = _make_deg()


BN = 1000


def _make_layer(do_ln: bool):
  def body(*refs):
    if do_ln:
      sum_ref, deg_ref, h_ref, wl_ref, bl_ref, wr_ref, g_ref, b_ref, o_ref = refs
    else:
      sum_ref, deg_ref, h_ref, wl_ref, bl_ref, wr_ref, o_ref = refs
    mean = sum_ref[...] / jnp.clip(deg_ref[...], 1.0, None)
    out = (jnp.dot(mean, wl_ref[...], preferred_element_type=jnp.float32)
           + bl_ref[...]
           + jnp.dot(h_ref[...], wr_ref[...], preferred_element_type=jnp.float32))
    out = jnp.maximum(out, 0.0)
    if do_ln:
      mu = jnp.mean(out, axis=1, keepdims=True)
      var = jnp.mean((out - mu) ** 2, axis=1, keepdims=True)
      out = (out - mu) / jnp.sqrt(var + 1e-5) * g_ref[...] + b_ref[...]
    o_ref[...] = out

  in_specs = [
      pl.BlockSpec((BN, D), lambda i: (i, 0)),
      pl.BlockSpec((BN, 1), lambda i: (i, 0)),
      pl.BlockSpec((BN, D), lambda i: (i, 0)),
      pl.BlockSpec((D, D), lambda i: (0, 0)),
      pl.BlockSpec((1, D), lambda i: (0, 0)),
      pl.BlockSpec((D, D), lambda i: (0, 0)),
  ]
  if do_ln:
    in_specs += [pl.BlockSpec((1, D), lambda i: (0, 0))] * 2
  return pl.pallas_call(
      body,
      grid=(N // BN,),
      in_specs=in_specs,
      out_specs=pl.BlockSpec((BN, D), lambda i: (i, 0)),
      out_shape=jax.ShapeDtypeStruct((N, D), jnp.float32),
  )


_layer_plain = _make_layer(False)
_layer_ln = _make_layer(True)


def kernel(x, edge_index, Wl1, bl1, Wr1, Wl2, bl2, Wr2, ln_g, ln_b):
  src = edge_index[0].astype(jnp.int32).reshape(NS, NBLK, K)
  dst = edge_index[1].astype(jnp.int32).reshape(NS, NBLK, K)
  deg2 = _deg(dst).reshape(N, 1)
  sums1 = _agg(x, src, dst)
  h1 = _layer_plain(sums1, deg2, x, Wl1.T, bl1.reshape(1, D), Wr1.T)
  sums2 = _agg(h1, src, dst)
  return _layer_ln(sums2, deg2, h1, Wl2.T, bl2.reshape(1, D), Wr2.T,
                   ln_g.reshape(1, D), ln_b.reshape(1, D))


# speedup vs baseline: 1.4182x; 1.3182x over previous
import jax
import jax.numpy as jnp
from jax import lax
from jax.experimental import pallas as pl
from jax.experimental.pallas import tpu as pltpu
from jax.experimental.pallas import tpu_sc as plsc

N = 10000
D = 128
E = 320000
NC = 2
NS = 16
HALF = N // NC
EPT = E // NS
K = 80
NBLK = EPT // K
ACC_ROWS = 5128
STRIPE = 320
ZR = 80
TRASH = HALF
DEG_ROWS = 5376
DEG_PAD = 5120


def _remap_dst(dst_s, c, s):
  lo = c * HALF
  trash_base = TRASH + s * 8

  def remap(i, carry):
    for j in range(K // 16):
      v = dst_s[i, pl.ds(j * 16, 16)]
      local = v - lo
      owned = (local >= 0) & (local < HALF)
      trash = trash_base + (v & 7)
      dst_s[i, pl.ds(j * 16, 16)] = jnp.where(owned, local, trash)
    return carry
  lax.fori_loop(0, NBLK, remap, 0)


DH = D // NC
ACC2_ROWS = 10240
STRIPE2 = 640
ZR2 = 128


def _make_agg():
  mesh = plsc.VectorSubcoreMesh(core_axis_name="c", subcore_axis_name="s")

  def body(h_hbm, src_hbm, dst_hbm, out_sum, src_s, dst_s, rows0, zbuf,
           acc_sh, sem_a):
    c = lax.axis_index("c")
    s = lax.axis_index("s")

    def zero_zbuf(i, carry):
      for j in range(DH // 16):
        zbuf[i, pl.ds(j * 16, 16)] = jnp.zeros((16,), jnp.float32)
      return carry
    lax.fori_loop(0, ZR2, zero_zbuf, 0)
    for r in range(STRIPE2 // ZR2):
      pltpu.sync_copy(zbuf, acc_sh.at[pl.ds(s * STRIPE2 + r * ZR2, ZR2)])

    pltpu.sync_copy(src_hbm.at[s], src_s)
    pltpu.sync_copy(dst_hbm.at[s], dst_s)

    def remap_src(i, carry):
      for j in range(K // 16):
        v = src_s[i, pl.ds(j * 16, 16)]
        src_s[i, pl.ds(j * 16, 16)] = v * 2 + c
      return carry
    lax.fori_loop(0, NBLK, remap_src, 0)
    plsc.subcore_barrier()

    def step(i, carry):
      pltpu.async_copy(h_hbm.at[src_s.at[i]], rows0, sem_a).wait()
      pltpu.sync_copy(rows0, acc_sh.at[dst_s.at[i]], add=True)
      return carry
    lax.fori_loop(0, NBLK, step, 0)

    plsc.subcore_barrier()
    @pl.when(s < NS - 1)
    def _():
      pltpu.sync_copy(acc_sh.at[pl.ds(s * STRIPE2, STRIPE2)],
                      out_sum.at[c, pl.ds(s * STRIPE2, STRIPE2)])

    @pl.when(s == NS - 1)
    def _():
      last = N - (NS - 1) * STRIPE2
      pltpu.sync_copy(acc_sh.at[pl.ds((NS - 1) * STRIPE2, last)],
                      out_sum.at[c, pl.ds((NS - 1) * STRIPE2, last)])

  return pl.kernel(
      body,
      out_type=jax.ShapeDtypeStruct((NC, N, DH), jnp.float32),
      mesh=mesh,
      compiler_params=pltpu.CompilerParams(use_tc_tiling_on_sc=False),
      scratch_types=[
          pltpu.VMEM((NBLK, K), jnp.int32),
          pltpu.VMEM((NBLK, K), jnp.int32),
          pltpu.VMEM((K, DH), jnp.float32),
          pltpu.VMEM((ZR2, DH), jnp.float32),
          pltpu.VMEM_SHARED((ACC2_ROWS, DH), jnp.float32),
          pltpu.SemaphoreType.DMA,
      ])


def _make_deg():
  mesh = plsc.VectorSubcoreMesh(core_axis_name="c", subcore_axis_name="s")

  def body(dst_hbm, out_deg, dst_s, ones_s, zdeg_s, deg_sh):
    c = lax.axis_index("c")
    s = lax.axis_index("s")
    lo = c * HALF

    def zero_zdeg(i, carry):
      zdeg_s[pl.ds(i * 16, 16)] = jnp.zeros((16,), jnp.float32)
      return carry
    lax.fori_loop(0, DEG_ROWS // 16, zero_zdeg, 0)
    for j in range(K // 16):
      ones_s[pl.ds(j * 16, 16)] = jnp.ones((16,), jnp.float32)

    @pl.when(s == 0)
    def _():
      pltpu.sync_copy(zdeg_s, deg_sh)

    pltpu.sync_copy(dst_hbm.at[s], dst_s)
    _remap_dst(dst_s, c, s)
    plsc.subcore_barrier()

    def step(i, carry):
      pltpu.sync_copy(ones_s, deg_sh.at[dst_s.at[i]], add=True)
      return carry
    lax.fori_loop(0, NBLK, step, 0)

    plsc.subcore_barrier()

    @pl.when(s == 0)
    def _():
      pltpu.sync_copy(deg_sh.at[pl.ds(0, HALF)], zdeg_s.at[pl.ds(0, HALF)])
      pltpu.sync_copy(zdeg_s.at[pl.ds(0, HALF)], out_deg.at[pl.ds(lo, HALF)])

  return pl.kernel(
      body,
      out_type=jax.ShapeDtypeStruct((N,), jnp.float32),
      mesh=mesh,
      scratch_types=[
          pltpu.VMEM((NBLK, K), jnp.int32),
          pltpu.VMEM((K,), jnp.float32),
          pltpu.VMEM((DEG_ROWS,), jnp.float32),
          pltpu.VMEM_SHARED((DEG_ROWS,), jnp.float32),
      ])


_agg = _make_agg()
_deg = _make_deg()


BN = 1000


def _make_layer(do_ln: bool):
  def body(*refs):
    if do_ln:
      sum_ref, deg_ref, h_ref, wl_ref, bl_ref, wr_ref, g_ref, b_ref, o_ref = refs
    else:
      sum_ref, deg_ref, h_ref, wl_ref, bl_ref, wr_ref, o_ref = refs
    ssum = jnp.concatenate([sum_ref[0], sum_ref[1]], axis=1)
    mean = ssum / jnp.clip(deg_ref[...], 1.0, None)
    out = (jnp.dot(mean, wl_ref[...], preferred_element_type=jnp.float32)
           + bl_ref[...]
           + jnp.dot(h_ref[...], wr_ref[...], preferred_element_type=jnp.float32))
    out = jnp.maximum(out, 0.0)
    if do_ln:
      mu = jnp.mean(out, axis=1, keepdims=True)
      var = jnp.mean((out - mu) ** 2, axis=1, keepdims=True)
      out = (out - mu) / jnp.sqrt(var + 1e-5) * g_ref[...] + b_ref[...]
    o_ref[...] = out

  in_specs = [
      pl.BlockSpec((NC, BN, D // NC), lambda i: (0, i, 0)),
      pl.BlockSpec((BN, 1), lambda i: (i, 0)),
      pl.BlockSpec((BN, D), lambda i: (i, 0)),
      pl.BlockSpec((D, D), lambda i: (0, 0)),
      pl.BlockSpec((1, D), lambda i: (0, 0)),
      pl.BlockSpec((D, D), lambda i: (0, 0)),
  ]
  if do_ln:
    in_specs += [pl.BlockSpec((1, D), lambda i: (0, 0))] * 2
  return pl.pallas_call(
      body,
      grid=(N // BN,),
      in_specs=in_specs,
      out_specs=pl.BlockSpec((BN, D), lambda i: (i, 0)),
      out_shape=jax.ShapeDtypeStruct((N, D), jnp.float32),
  )


_layer_plain = _make_layer(False)
_layer_ln = _make_layer(True)


def kernel(x, edge_index, Wl1, bl1, Wr1, Wl2, bl2, Wr2, ln_g, ln_b):
  src = edge_index[0].astype(jnp.int32).reshape(NS, NBLK, K)
  dst = edge_index[1].astype(jnp.int32).reshape(NS, NBLK, K)
  deg2 = _deg(dst).reshape(N, 1)
  sums1 = _agg(x.reshape(2 * N, D // NC), src, dst)
  h1 = _layer_plain(sums1, deg2, x, Wl1.T, bl1.reshape(1, D), Wr1.T)
  sums2 = _agg(h1.reshape(2 * N, D // NC), src, dst)
  return _layer_ln(sums2, deg2, h1, Wl2.T, bl2.reshape(1, D), Wr2.T,
                   ln_g.reshape(1, D), ln_b.reshape(1, D))
